# Initial kernel scaffold; baseline (speedup 1.0000x reference)
import jax
import jax.numpy as jnp
from jax.experimental import pallas as pl


def kernel(x, pos, edge_index, edge_attr, rad0_W1, rad0_b1, rad0_W2, rad0_b2, si0_0, radL_W1, radL_b1, radL_W2, radL_b2, siL_0, siL_1):
    raise NotImplementedError("write your pallas kernel here")


# speedup vs baseline: 3.0549x; 3.0549x over previous
import jax
import jax.numpy as jnp
from jax import lax
from jax.experimental import pallas as pl
from jax.experimental.pallas import tpu as pltpu
from jax.experimental.pallas import tpu_sc as plsc

N = 10000
E = 320000
D = 128
EDIM = 16
HID = 32

NC = 2
NS = 16
NW = NC * NS
EPW = E // NW
CH = 80
NCH = EPW // CH
PAD = 40960

_f32 = jnp.float32
_i32 = jnp.int32

_MESH = dict(core_axis_name="c", subcore_axis_name="s")


def _prep_body(x_ref, pos_ref, wz_ref, ws_ref, ztab_ref, ptab_ref, prev0_ref):
    xb = x_ref[...]
    nb = xb.shape[0]
    z = jnp.dot(xb, wz_ref[...], preferred_element_type=_f32)
    y2 = jnp.dot(xb, ws_ref[...], preferred_element_type=_f32)
    posb = pos_ref[...]
    ztab_ref[...] = jnp.concatenate(
        [z, posb, y2[:, :2], jnp.zeros((nb, 59), _f32)], axis=1)
    ptab_ref[...] = jnp.concatenate([posb, y2[:, :5]], axis=1)
    prev0_ref[...] = jnp.concatenate(
        [y2[:, 2:3], jnp.zeros((nb, 3), _f32)], axis=1)


def _tc_prep(x, pos, wz, ws):
    blk = 2000
    grid = (N // blk,)
    return pl.pallas_call(
        _prep_body,
        grid=grid,
        in_specs=[
            pl.BlockSpec((blk, D), lambda i: (i, 0)),
            pl.BlockSpec((blk, 3), lambda i: (i, 0)),
            pl.BlockSpec((D, 64), lambda i: (0, 0)),
            pl.BlockSpec((D, 8), lambda i: (0, 0)),
        ],
        out_specs=[
            pl.BlockSpec((blk, 128), lambda i: (i, 0)),
            pl.BlockSpec((blk, 8), lambda i: (i, 0)),
            pl.BlockSpec((blk, 4), lambda i: (i, 0)),
        ],
        out_shape=[
            jax.ShapeDtypeStruct((N, 128), _f32),
            jax.ShapeDtypeStruct((N, 8), _f32),
            jax.ShapeDtypeStruct((N, 4), _f32),
        ],
    )(x, pos, wz, ws)


def _sc_gather0_body(src_hbm, pdidx_hbm, ztab_hbm, ptabf_hbm,
                     zsrc_out, pd_out,
                     src_v, pdidx_v, zbuf, pdbuf, sem1, sem2):
    c = lax.axis_index("c")
    s = lax.axis_index("s")
    base = (s * NC + c) * EPW

    def chunk(j, carry):
        off = pl.multiple_of(base + j * CH, 16)
        off4 = pl.multiple_of(off * 4, 64)
        pltpu.sync_copy(src_hbm.at[pl.ds(off, CH)], src_v)
        pltpu.sync_copy(pdidx_hbm.at[pl.ds(off4, CH * 4)], pdidx_v)
        cp1 = pltpu.async_copy(ztab_hbm.at[src_v], zbuf, sem1)
        cp2 = pltpu.async_copy(ptabf_hbm.at[pdidx_v], pdbuf, sem2)
        cp1.wait()
        cp2.wait()
        pltpu.sync_copy(zbuf, zsrc_out.at[pl.ds(off, CH), :])
        pltpu.sync_copy(pdbuf, pd_out.at[pl.ds(off4, CH * 4)])
        return carry

    lax.fori_loop(0, NCH, chunk, 0)


def _sc_gather0(src, pdidx_f, ztab, ptab_f):
    return pl.kernel(
        _sc_gather0_body,
        out_type=(
            jax.ShapeDtypeStruct((E, 128), _f32),
            jax.ShapeDtypeStruct((E * 4,), _f32),
        ),
        mesh=plsc.VectorSubcoreMesh(**_MESH),
        scratch_types=[
            pltpu.VMEM((CH,), _i32),
            pltpu.VMEM((CH * 4,), _i32),
            pltpu.VMEM((CH, 128), _f32),
            pltpu.VMEM((CH * 4,), _f32),
            pltpu.SemaphoreType.DMA,
            pltpu.SemaphoreType.DMA,
        ],
    )(src, pdidx_f, ztab, ptab_f)


def _tc0_body(src_ref, dst_ref, zs_ref, pd_ref, ea_ref,
              w1r2_ref, w1e2_ref, b12_ref, msel_ref,
              w1Lr_ref, w1Le_ref, b1L_ref, w2L_ref, b2L_ref,
              a0_ref, d4_ref, gidx_ref, sidx_ref,
              wA1_ref, wB1_ref, wA2_ref, wB2_ref, wA3_ref, wB3_ref):
    zs = zs_ref[...]
    pd = pd_ref[...]
    rel = pd[:, 0:3] - zs[:, 64:67]
    c0s = zs[:, 67:68]
    c1s = zs[:, 68:69]
    ea = ea_ref[...]
    r2 = jnp.sum(rel * rel, axis=-1, keepdims=True)
    r = jnp.sqrt(r2)
    dirn = rel / (r + 1e-8)
    nb = rel.shape[0]

    hid2 = jnp.maximum(
        r * w1r2_ref[...] + jnp.dot(ea, w1e2_ref[...],
                                    preferred_element_type=_f32)
        + b12_ref[...], 0.0)
    prod = hid2 * zs[:, 0:64]
    ms = jnp.dot(prod, msel_ref[...], preferred_element_type=_f32)
    m0 = ms[:, 0:1] + c0s
    s01 = ms[:, 1:2] + c1s
    a0_ref[...] = jnp.concatenate([m0, s01 * dirn], axis=1)

    d4_ref[...] = jnp.concatenate([jnp.zeros((nb, 1), _f32), dirn], axis=1)
    lane = lax.broadcasted_iota(_i32, (nb, 4), 1)
    gidx_ref[...] = src_ref[0, 0, :].reshape(nb, 1) * 4 + lane
    sidx_ref[...] = dst_ref[0, 0, :].reshape(nb, 1) * 4 + lane

    for i, (wA_ref, wB_ref) in enumerate(
            ((wA1_ref, wB1_ref), (wA2_ref, wB2_ref), (wA3_ref, wB3_ref))):
        hidL = jnp.maximum(
            r * w1Lr_ref[i][None, :]
            + jnp.dot(ea, w1Le_ref[i], preferred_element_type=_f32)
            + b1L_ref[i][None, :], 0.0)
        feat = (jnp.dot(hidL, w2L_ref[i], preferred_element_type=_f32)
                + b2L_ref[i][None, :])
        f00 = feat[:, 0:1]
        f01 = feat[:, 1:2]
        f10 = feat[:, 2:3]
        f11 = feat[:, 3:4]
        wA_ref[...] = jnp.concatenate([f00, f11, f11, f11], axis=1)
        wB_ref[...] = jnp.concatenate([f10, f01, f01, f01], axis=1)


def _tc0(src3, dst3, zsrc, pd4, ea,
         w1r2, w1e2, b12, msel, w1Lr, w1Le, b1L, w2L, b2L):
    blk = 2560
    grid = (E // blk,)
    full = lambda *dims: pl.BlockSpec(dims, lambda i: tuple(0 for _ in dims))
    eblk4 = pl.BlockSpec((blk, 4), lambda i: (i, 0))
    out4 = jax.ShapeDtypeStruct((E, 4), _f32)
    out4i = jax.ShapeDtypeStruct((E, 4), _i32)
    return pl.pallas_call(
        _tc0_body,
        grid=grid,
        in_specs=[
            pl.BlockSpec((1, 1, blk), lambda i: (i, 0, 0)),
            pl.BlockSpec((1, 1, blk), lambda i: (i, 0, 0)),
            pl.BlockSpec((blk, 128), lambda i: (i, 0)),
            pl.BlockSpec((blk, 4), lambda i: (i, 0)),
            pl.BlockSpec((blk, EDIM), lambda i: (i, 0)),
            full(1, 64), full(EDIM, 64), full(1, 64),
            full(64, 8),
            full(3, HID), full(3, EDIM, HID), full(3, HID),
            full(3, HID, 4), full(3, 4),
        ],
        out_specs=[eblk4] * 10,
        out_shape=[out4, out4, out4i, out4i] + [out4] * 6,
    )(src3, dst3, zsrc, pd4, ea,
      w1r2, w1e2, b12, msel, w1Lr, w1Le, b1L, w2L, b2L)


def _sc_scatter0_body(sidx_hbm, a0_hbm, zeros_hbm, part_out,
                      dst_v, msg_v, acc):
    c = lax.axis_index("c")
    s = lax.axis_index("s")
    base4 = (s * NC + c) * (EPW * 4)
    soff = pl.multiple_of(s * PAD, 128)

    pltpu.sync_copy(zeros_hbm, acc.at[pl.ds(soff, PAD)])

    def chunk(j, carry):
        off4 = pl.multiple_of(base4 + j * (CH * 4), 64)
        pltpu.sync_copy(sidx_hbm.at[pl.ds(off4, CH * 4)], dst_v)
        pltpu.sync_copy(a0_hbm.at[pl.ds(off4, CH * 4)], msg_v)
        for t in range(CH * 4 // 16):
            o = t * 16
            dst_v[pl.ds(o, 16)] = dst_v[pl.ds(o, 16)] + soff
        pltpu.sync_copy(msg_v, acc.at[dst_v], add=True)
        return carry

    lax.fori_loop(0, NCH, chunk, 0)
    pltpu.sync_copy(acc.at[pl.ds(soff, PAD)],
                    part_out.at[c, pl.ds(soff, PAD)])


def _sc_scatter0(sidx_f, a0_f, zeros_pad):
    return pl.kernel(
        _sc_scatter0_body,
        out_type=jax.ShapeDtypeStruct((NC, NS * PAD), _f32),
        mesh=plsc.VectorSubcoreMesh(**_MESH),
        scratch_types=[
            pltpu.VMEM((CH * 4,), _i32),
            pltpu.VMEM((CH * 4,), _f32),
            pltpu.VMEM_SHARED((NS * PAD,), _f32),
        ],
    )(sidx_f, a0_f, zeros_pad)


def _sc_mid_body(gidx_hbm, sidx_hbm, d4_hbm, wA_hbm, wB_hbm, t4_hbm,
                 zeros_hbm, part_out,
                 gidx_v, sidx_v, d4_v, wA_v, wB_v, hs_v, msg_v, acc, sem):
    c = lax.axis_index("c")
    s = lax.axis_index("s")
    base4 = (s * NC + c) * (EPW * 4)
    soff = pl.multiple_of(s * PAD, 128)

    pltpu.sync_copy(zeros_hbm, acc.at[pl.ds(soff, PAD)])

    lanes = lax.iota(_i32, 16)
    e0 = (lanes & 3) == 0
    permb = lanes ^ 1
    permq = lanes ^ 2
    permh = (lanes >> 2) * 4

    def chunk(j, carry):
        off4 = pl.multiple_of(base4 + j * (CH * 4), 64)
        pltpu.sync_copy(gidx_hbm.at[pl.ds(off4, CH * 4)], gidx_v)
        pltpu.sync_copy(sidx_hbm.at[pl.ds(off4, CH * 4)], sidx_v)
        pltpu.sync_copy(d4_hbm.at[pl.ds(off4, CH * 4)], d4_v)
        pltpu.sync_copy(wA_hbm.at[pl.ds(off4, CH * 4)], wA_v)
        pltpu.sync_copy(wB_hbm.at[pl.ds(off4, CH * 4)], wB_v)
        pltpu.async_copy(t4_hbm.at[gidx_v], hs_v, sem).wait()
        for t in range(CH * 4 // 16):
            o = t * 16
            hs = hs_v[pl.ds(o, 16)]
            d4 = d4_v[pl.ds(o, 16)]
            wa = wA_v[pl.ds(o, 16)]
            wb = wB_v[pl.ds(o, 16)]
            p = d4 * hs
            q = p + p[permb]
            dotv = q + q[permq]
            h0b = hs[permh]
            m4 = jnp.where(e0, dotv, h0b * d4)
            msg_v[pl.ds(o, 16)] = wa * hs + wb * m4
            sidx_v[pl.ds(o, 16)] = sidx_v[pl.ds(o, 16)] + soff
        pltpu.sync_copy(msg_v, acc.at[sidx_v], add=True)
        return carry

    lax.fori_loop(0, NCH, chunk, 0)
    pltpu.sync_copy(acc.at[pl.ds(soff, PAD)],
                    part_out.at[c, pl.ds(soff, PAD)])


def _sc_mid(gidx, sidx, d4, wA, wB, t4, zeros_pad):
    return pl.kernel(
        _sc_mid_body,
        out_type=jax.ShapeDtypeStruct((NC, NS * PAD), _f32),
        mesh=plsc.VectorSubcoreMesh(**_MESH),
        scratch_types=[
            pltpu.VMEM((CH * 4,), _i32),
            pltpu.VMEM((CH * 4,), _i32),
            pltpu.VMEM((CH * 4,), _f32),
            pltpu.VMEM((CH * 4,), _f32),
            pltpu.VMEM((CH * 4,), _f32),
            pltpu.VMEM((CH * 4,), _f32),
            pltpu.VMEM((CH * 4,), _f32),
            pltpu.VMEM_SHARED((NS * PAD,), _f32),
            pltpu.SemaphoreType.DMA,
        ],
    )(gidx, sidx, d4, wA, wB, t4, zeros_pad)


def _fin_body(p_ref, prev_ref, si_ref, out_ref):
    out_ref[...] = (jnp.sum(p_ref[...], axis=0)
                    + prev_ref[...] * si_ref[...])


def _fin(part, prev, si16):
    rows = N // 4
    p32 = part.reshape(NW, PAD)[:, :N * 4].reshape(NW, rows, 16)
    return pl.pallas_call(
        _fin_body,
        out_shape=jax.ShapeDtypeStruct((rows, 16), _f32),
    )(p32, prev.reshape(rows, 16), si16)


def _jnp_gather0(src, pdidx_f, ztab, ptab_f):
    return ztab[src], ptab_f[pdidx_f]


def _jnp_scatter0(dst, a0, zeros_nt):
    from jax.ops import segment_sum
    seg = segment_sum(a0, dst, num_segments=N)
    return jnp.stack([seg, jnp.zeros_like(seg)])


def _jnp_mid(gidx, sidx, d4, wA, wB, t4, zeros_flat):
    hs = t4[gidx].reshape(E, 4)
    d = d4.reshape(E, 4)
    wa = wA.reshape(E, 4)
    wb = wB.reshape(E, 4)
    dot = (d * hs).sum(-1, keepdims=True)
    m4 = jnp.concatenate([dot, hs[:, 0:1] * d[:, 1:]], axis=1)
    msg = (wa * hs + wb * m4).reshape(-1)
    accf = jnp.zeros((N * 4,), _f32).at[sidx].add(msg)
    return jnp.stack([accf, jnp.zeros_like(accf)])


def kernel(x, pos, edge_index, edge_attr, rad0_W1, rad0_b1, rad0_W2,
           rad0_b2, si0_0, radL_W1, radL_b1, radL_W2, radL_b2, siL_0, siL_1):
    src = edge_index[0]
    dst = edge_index[1]
    blk = 2560
    src3 = src.reshape(E // blk, 1, blk)
    dst3 = dst.reshape(E // blk, 1, blk)

    wz = jnp.concatenate([rad0_W2[:, :D].T, rad0_W2[:, D:].T], axis=1)
    ws = jnp.stack(
        [rad0_b2[:D], rad0_b2[D:], si0_0[0]]
        + [jnp.zeros((D,), _f32)] * 5, axis=1)
    w1r2 = jnp.concatenate([rad0_W1[0:1], rad0_W1[0:1]], axis=1)
    w1e2 = jnp.concatenate([rad0_W1[1:], rad0_W1[1:]], axis=1)
    b12 = jnp.concatenate([rad0_b1, rad0_b1]).reshape(1, 64)
    eye2 = jnp.zeros((64, 8), _f32)
    eye2 = eye2.at[:HID, 0].set(1.0).at[HID:, 1].set(1.0)
    w1Lr = radL_W1[:, 0, :]
    w1Le = radL_W1[:, 1:, :]
    zeros_pad = jnp.zeros((PAD,), _f32)

    def sivec(i):
        v4 = jnp.concatenate([siL_0[i, 0], siL_1[i, 0], siL_1[i, 0],
                              siL_1[i, 0]])
        return jnp.tile(v4, 4).reshape(1, 16)

    ztab, ptab, prev0 = _tc_prep(x, pos, wz, ws)
    pdidx_f = (dst[:, None] * 8 + jnp.arange(4, dtype=_i32)).reshape(E * 4)
    zsrc, pd_f = _sc_gather0(src, pdidx_f, ztab, ptab.reshape(N * 8))
    (a0, d4, gidx, sidx, wA1, wB1, wA2, wB2, wA3, wB3) = _tc0(
        src3, dst3, zsrc, pd_f.reshape(E, 4), edge_attr,
        w1r2, w1e2, b12, eye2, w1Lr, w1Le, radL_b1, radL_W2, radL_b2)
    gidx_f = gidx.reshape(E * 4)
    sidx_f = sidx.reshape(E * 4)
    d4_f = d4.reshape(E * 4)
    part0 = _sc_scatter0(sidx_f, a0.reshape(E * 4), zeros_pad)
    ones16 = jnp.ones((1, 16), _f32)
    t1 = _fin(part0, prev0, ones16)
    part1 = _sc_mid(gidx_f, sidx_f, d4_f, wA1.reshape(E * 4),
                    wB1.reshape(E * 4), t1.reshape(N * 4), zeros_pad)
    t2 = _fin(part1, t1, sivec(0))
    part2 = _sc_mid(gidx_f, sidx_f, d4_f, wA2.reshape(E * 4),
                    wB2.reshape(E * 4), t2.reshape(N * 4), zeros_pad)
    t3 = _fin(part2, t2, sivec(1))
    part3 = _sc_mid(gidx_f, sidx_f, d4_f, wA3.reshape(E * 4),
                    wB3.reshape(E * 4), t3.reshape(N * 4), zeros_pad)
    out = _fin(part3, t3, sivec(2))
    return out.reshape(N, 4)


# speedup vs baseline: 3.7070x; 1.2135x over previous
import jax
import jax.numpy as jnp
from jax import lax
from jax.experimental import pallas as pl
from jax.experimental.pallas import tpu as pltpu
from jax.experimental.pallas import tpu_sc as plsc

N = 10000
E = 320000
D = 128
EDIM = 16
HID = 32

NC = 2
NS = 16
NW = NC * NS
EPW = E // NW
CH = 80
NCH = EPW // CH
PAD = 40960

_f32 = jnp.float32
_i32 = jnp.int32

_MESH = dict(core_axis_name="c", subcore_axis_name="s")


def _prep_body(x_ref, pos_ref, wz_ref, ws_ref, ztab_ref, ptab_ref, prev0_ref):
    xb = x_ref[...]
    nb = xb.shape[0]
    z = jnp.dot(xb, wz_ref[...], preferred_element_type=_f32)
    y2 = jnp.dot(xb, ws_ref[...], preferred_element_type=_f32)
    posb = pos_ref[...]
    ztab_ref[...] = jnp.concatenate(
        [z, posb, y2[:, :2], jnp.zeros((nb, 59), _f32)], axis=1)
    ptab_ref[...] = jnp.concatenate([posb, y2[:, :5]], axis=1)
    prev0_ref[...] = jnp.concatenate(
        [y2[:, 2:3], jnp.zeros((nb, 3), _f32)], axis=1)


def _tc_prep(x, pos, wz, ws):
    blk = 2000
    grid = (N // blk,)
    return pl.pallas_call(
        _prep_body,
        grid=grid,
        in_specs=[
            pl.BlockSpec((blk, D), lambda i: (i, 0)),
            pl.BlockSpec((blk, 3), lambda i: (i, 0)),
            pl.BlockSpec((D, 64), lambda i: (0, 0)),
            pl.BlockSpec((D, 8), lambda i: (0, 0)),
        ],
        out_specs=[
            pl.BlockSpec((blk, 128), lambda i: (i, 0)),
            pl.BlockSpec((blk, 8), lambda i: (i, 0)),
            pl.BlockSpec((blk, 4), lambda i: (i, 0)),
        ],
        out_shape=[
            jax.ShapeDtypeStruct((N, 128), _f32),
            jax.ShapeDtypeStruct((N, 8), _f32),
            jax.ShapeDtypeStruct((N, 4), _f32),
        ],
    )(x, pos, wz, ws)


def _sc_gather0_body(src_hbm, pdidx_hbm, ztab_hbm, ptabf_hbm,
                     zsrc_out, pd_out,
                     src_v, pdidx_v, zbuf, pdbuf, sem1, sem2, sem3, sem4):
    c = lax.axis_index("c")
    s = lax.axis_index("s")
    base = (s * NC + c) * EPW

    def chunk(j, carry):
        off = pl.multiple_of(base + j * CH, 16)
        off4 = pl.multiple_of(off * 4, 64)
        ld1 = pltpu.async_copy(src_hbm.at[pl.ds(off, CH)], src_v, sem3)
        ld2 = pltpu.async_copy(pdidx_hbm.at[pl.ds(off4, CH * 4)], pdidx_v,
                               sem4)
        ld1.wait()
        cp1 = pltpu.async_copy(ztab_hbm.at[src_v], zbuf, sem1)
        ld2.wait()
        cp2 = pltpu.async_copy(ptabf_hbm.at[pdidx_v], pdbuf, sem2)
        cp1.wait()
        st1 = pltpu.async_copy(zbuf, zsrc_out.at[pl.ds(off, CH), :], sem3)
        cp2.wait()
        st2 = pltpu.async_copy(pdbuf, pd_out.at[pl.ds(off4, CH * 4)], sem4)
        st1.wait()
        st2.wait()
        return carry

    lax.fori_loop(0, NCH, chunk, 0)


def _sc_gather0(src, pdidx_f, ztab, ptab_f):
    return pl.kernel(
        _sc_gather0_body,
        out_type=(
            jax.ShapeDtypeStruct((E, 128), _f32),
            jax.ShapeDtypeStruct((E * 4,), _f32),
        ),
        mesh=plsc.VectorSubcoreMesh(**_MESH),
        scratch_types=[
            pltpu.VMEM((CH,), _i32),
            pltpu.VMEM((CH * 4,), _i32),
            pltpu.VMEM((CH, 128), _f32),
            pltpu.VMEM((CH * 4,), _f32),
            pltpu.SemaphoreType.DMA,
            pltpu.SemaphoreType.DMA,
            pltpu.SemaphoreType.DMA,
            pltpu.SemaphoreType.DMA,
        ],
    )(src, pdidx_f, ztab, ptab_f)


def _tc0_body(src_ref, dst_ref, zs_ref, pd_ref, ea_ref,
              w1r2_ref, w1e2_ref, b12_ref, msel_ref,
              w1Lr_ref, w1Le_ref, b1L_ref, w2L_ref, b2L_ref,
              a0_ref, d4_ref, gidx_ref, sidx_ref,
              wA1_ref, wB1_ref, wA2_ref, wB2_ref, wA3_ref, wB3_ref):
    zs = zs_ref[...]
    pd = pd_ref[...]
    rel = pd[:, 0:3] - zs[:, 64:67]
    c0s = zs[:, 67:68]
    c1s = zs[:, 68:69]
    ea = ea_ref[...]
    r2 = jnp.sum(rel * rel, axis=-1, keepdims=True)
    r = jnp.sqrt(r2)
    dirn = rel / (r + 1e-8)
    nb = rel.shape[0]

    hid2 = jnp.maximum(
        r * w1r2_ref[...] + jnp.dot(ea, w1e2_ref[...],
                                    preferred_element_type=_f32)
        + b12_ref[...], 0.0)
    prod = hid2 * zs[:, 0:64]
    ms = jnp.dot(prod, msel_ref[...], preferred_element_type=_f32)
    m0 = ms[:, 0:1] + c0s
    s01 = ms[:, 1:2] + c1s
    a0_ref[...] = jnp.concatenate([m0, s01 * dirn], axis=1)

    d4_ref[...] = jnp.concatenate([jnp.zeros((nb, 1), _f32), dirn], axis=1)
    lane = lax.broadcasted_iota(_i32, (nb, 4), 1)
    gidx_ref[...] = src_ref[0, 0, :].reshape(nb, 1) * 4 + lane
    sidx_ref[...] = dst_ref[0, 0, :].reshape(nb, 1) * 4 + lane

    for i, (wA_ref, wB_ref) in enumerate(
            ((wA1_ref, wB1_ref), (wA2_ref, wB2_ref), (wA3_ref, wB3_ref))):
        hidL = jnp.maximum(
            r * w1Lr_ref[i][None, :]
            + jnp.dot(ea, w1Le_ref[i], preferred_element_type=_f32)
            + b1L_ref[i][None, :], 0.0)
        feat = (jnp.dot(hidL, w2L_ref[i], preferred_element_type=_f32)
                + b2L_ref[i][None, :])
        f00 = feat[:, 0:1]
        f01 = feat[:, 1:2]
        f10 = feat[:, 2:3]
        f11 = feat[:, 3:4]
        wA_ref[...] = jnp.concatenate([f00, f11, f11, f11], axis=1)
        wB_ref[...] = jnp.concatenate([f10, f01, f01, f01], axis=1)


def _tc0(src3, dst3, zsrc, pd4, ea,
         w1r2, w1e2, b12, msel, w1Lr, w1Le, b1L, w2L, b2L):
    blk = 2560
    grid = (E // blk,)
    full = lambda *dims: pl.BlockSpec(dims, lambda i: tuple(0 for _ in dims))
    eblk4 = pl.BlockSpec((blk, 4), lambda i: (i, 0))
    out4 = jax.ShapeDtypeStruct((E, 4), _f32)
    out4i = jax.ShapeDtypeStruct((E, 4), _i32)
    return pl.pallas_call(
        _tc0_body,
        grid=grid,
        in_specs=[
            pl.BlockSpec((1, 1, blk), lambda i: (i, 0, 0)),
            pl.BlockSpec((1, 1, blk), lambda i: (i, 0, 0)),
            pl.BlockSpec((blk, 128), lambda i: (i, 0)),
            pl.BlockSpec((blk, 4), lambda i: (i, 0)),
            pl.BlockSpec((blk, EDIM), lambda i: (i, 0)),
            full(1, 64), full(EDIM, 64), full(1, 64),
            full(64, 8),
            full(3, HID), full(3, EDIM, HID), full(3, HID),
            full(3, HID, 4), full(3, 4),
        ],
        out_specs=[eblk4] * 10,
        out_shape=[out4, out4, out4i, out4i] + [out4] * 6,
    )(src3, dst3, zsrc, pd4, ea,
      w1r2, w1e2, b12, msel, w1Lr, w1Le, b1L, w2L, b2L)


def _sc_scatter0_body(sidx_hbm, a0_hbm, zeros_hbm, part_out,
                      dst_v, msg_v, acc):
    c = lax.axis_index("c")
    s = lax.axis_index("s")
    base4 = (s * NC + c) * (EPW * 4)
    soff = pl.multiple_of(s * PAD, 128)

    pltpu.sync_copy(zeros_hbm, acc.at[pl.ds(soff, PAD)])

    def chunk(j, carry):
        off4 = pl.multiple_of(base4 + j * (CH * 4), 64)
        pltpu.sync_copy(sidx_hbm.at[pl.ds(off4, CH * 4)], dst_v)
        pltpu.sync_copy(a0_hbm.at[pl.ds(off4, CH * 4)], msg_v)
        for t in range(CH * 4 // 16):
            o = t * 16
            dst_v[pl.ds(o, 16)] = dst_v[pl.ds(o, 16)] + soff
        pltpu.sync_copy(msg_v, acc.at[dst_v], add=True)
        return carry

    lax.fori_loop(0, NCH, chunk, 0)
    pltpu.sync_copy(acc.at[pl.ds(soff, PAD)],
                    part_out.at[c, pl.ds(soff, PAD)])


def _sc_scatter0(sidx_f, a0_f, zeros_pad):
    return pl.kernel(
        _sc_scatter0_body,
        out_type=jax.ShapeDtypeStruct((NC, NS * PAD), _f32),
        mesh=plsc.VectorSubcoreMesh(**_MESH),
        scratch_types=[
            pltpu.VMEM((CH * 4,), _i32),
            pltpu.VMEM((CH * 4,), _f32),
            pltpu.VMEM_SHARED((NS * PAD,), _f32),
        ],
    )(sidx_f, a0_f, zeros_pad)


def _sc_mid_body(gidx_hbm, sidx_hbm, d4_hbm, wA_hbm, wB_hbm, t4_hbm,
                 zeros_hbm, part_out,
                 gidx_v, sidx_v, d4_v, wA_v, wB_v, hs_v, msg_v, acc, sem,
                 sem1, sem2, sem3, sem4, sem5):
    c = lax.axis_index("c")
    s = lax.axis_index("s")
    base4 = (s * NC + c) * (EPW * 4)
    soff = pl.multiple_of(s * PAD, 128)

    pltpu.sync_copy(zeros_hbm, acc.at[pl.ds(soff, PAD)])

    lanes = lax.iota(_i32, 16)
    e0 = (lanes & 3) == 0
    permb = lanes ^ 1
    permq = lanes ^ 2
    permh = (lanes >> 2) * 4

    def chunk(j, carry):
        off4 = pl.multiple_of(base4 + j * (CH * 4), 64)
        ld1 = pltpu.async_copy(gidx_hbm.at[pl.ds(off4, CH * 4)], gidx_v,
                               sem1)
        ld2 = pltpu.async_copy(sidx_hbm.at[pl.ds(off4, CH * 4)], sidx_v,
                               sem2)
        ld3 = pltpu.async_copy(d4_hbm.at[pl.ds(off4, CH * 4)], d4_v, sem3)
        ld4 = pltpu.async_copy(wA_hbm.at[pl.ds(off4, CH * 4)], wA_v, sem4)
        ld5 = pltpu.async_copy(wB_hbm.at[pl.ds(off4, CH * 4)], wB_v, sem5)
        ld1.wait()
        cpg = pltpu.async_copy(t4_hbm.at[gidx_v], hs_v, sem)
        ld2.wait()
        ld3.wait()
        ld4.wait()
        ld5.wait()
        cpg.wait()
        for t in range(CH * 4 // 16):
            o = t * 16
            hs = hs_v[pl.ds(o, 16)]
            d4 = d4_v[pl.ds(o, 16)]
            wa = wA_v[pl.ds(o, 16)]
            wb = wB_v[pl.ds(o, 16)]
            p = d4 * hs
            q = p + p[permb]
            dotv = q + q[permq]
            h0b = hs[permh]
            m4 = jnp.where(e0, dotv, h0b * d4)
            msg_v[pl.ds(o, 16)] = wa * hs + wb * m4
            sidx_v[pl.ds(o, 16)] = sidx_v[pl.ds(o, 16)] + soff
        pltpu.sync_copy(msg_v, acc.at[sidx_v], add=True)
        return carry

    lax.fori_loop(0, NCH, chunk, 0)
    pltpu.sync_copy(acc.at[pl.ds(soff, PAD)],
                    part_out.at[c, pl.ds(soff, PAD)])


def _sc_mid(gidx, sidx, d4, wA, wB, t4, zeros_pad):
    return pl.kernel(
        _sc_mid_body,
        out_type=jax.ShapeDtypeStruct((NC, NS * PAD), _f32),
        mesh=plsc.VectorSubcoreMesh(**_MESH),
        scratch_types=[
            pltpu.VMEM((CH * 4,), _i32),
            pltpu.VMEM((CH * 4,), _i32),
            pltpu.VMEM((CH * 4,), _f32),
            pltpu.VMEM((CH * 4,), _f32),
            pltpu.VMEM((CH * 4,), _f32),
            pltpu.VMEM((CH * 4,), _f32),
            pltpu.VMEM((CH * 4,), _f32),
            pltpu.VMEM_SHARED((NS * PAD,), _f32),
            pltpu.SemaphoreType.DMA,
            pltpu.SemaphoreType.DMA,
            pltpu.SemaphoreType.DMA,
            pltpu.SemaphoreType.DMA,
            pltpu.SemaphoreType.DMA,
            pltpu.SemaphoreType.DMA,
        ],
    )(gidx, sidx, d4, wA, wB, t4, zeros_pad)


def _fin_body(p_ref, prev_ref, si_ref, out_ref):
    out_ref[...] = (jnp.sum(p_ref[...], axis=0)
                    + prev_ref[...] * si_ref[...])


def _fin(part, prev, si16):
    rows = N // 4
    p32 = part.reshape(NW, PAD)[:, :N * 4].reshape(NW, rows, 16)
    return pl.pallas_call(
        _fin_body,
        out_shape=jax.ShapeDtypeStruct((rows, 16), _f32),
    )(p32, prev.reshape(rows, 16), si16)


def _jnp_gather0(src, pdidx_f, ztab, ptab_f):
    return ztab[src], ptab_f[pdidx_f]


def _jnp_scatter0(dst, a0, zeros_nt):
    from jax.ops import segment_sum
    seg = segment_sum(a0, dst, num_segments=N)
    return jnp.stack([seg, jnp.zeros_like(seg)])


def _jnp_mid(gidx, sidx, d4, wA, wB, t4, zeros_flat):
    hs = t4[gidx].reshape(E, 4)
    d = d4.reshape(E, 4)
    wa = wA.reshape(E, 4)
    wb = wB.reshape(E, 4)
    dot = (d * hs).sum(-1, keepdims=True)
    m4 = jnp.concatenate([dot, hs[:, 0:1] * d[:, 1:]], axis=1)
    msg = (wa * hs + wb * m4).reshape(-1)
    accf = jnp.zeros((N * 4,), _f32).at[sidx].add(msg)
    return jnp.stack([accf, jnp.zeros_like(accf)])


def kernel(x, pos, edge_index, edge_attr, rad0_W1, rad0_b1, rad0_W2,
           rad0_b2, si0_0, radL_W1, radL_b1, radL_W2, radL_b2, siL_0, siL_1):
    src = edge_index[0]
    dst = edge_index[1]
    blk = 2560
    src3 = src.reshape(E // blk, 1, blk)
    dst3 = dst.reshape(E // blk, 1, blk)

    wz = jnp.concatenate([rad0_W2[:, :D].T, rad0_W2[:, D:].T], axis=1)
    ws = jnp.stack(
        [rad0_b2[:D], rad0_b2[D:], si0_0[0]]
        + [jnp.zeros((D,), _f32)] * 5, axis=1)
    w1r2 = jnp.concatenate([rad0_W1[0:1], rad0_W1[0:1]], axis=1)
    w1e2 = jnp.concatenate([rad0_W1[1:], rad0_W1[1:]], axis=1)
    b12 = jnp.concatenate([rad0_b1, rad0_b1]).reshape(1, 64)
    eye2 = jnp.zeros((64, 8), _f32)
    eye2 = eye2.at[:HID, 0].set(1.0).at[HID:, 1].set(1.0)
    w1Lr = radL_W1[:, 0, :]
    w1Le = radL_W1[:, 1:, :]
    zeros_pad = jnp.zeros((PAD,), _f32)

    def sivec(i):
        v4 = jnp.concatenate([siL_0[i, 0], siL_1[i, 0], siL_1[i, 0],
                              siL_1[i, 0]])
        return jnp.tile(v4, 4).reshape(1, 16)

    ztab, ptab, prev0 = _tc_prep(x, pos, wz, ws)
    pdidx_f = (dst[:, None] * 8 + jnp.arange(4, dtype=_i32)).reshape(E * 4)
    zsrc, pd_f = _sc_gather0(src, pdidx_f, ztab, ptab.reshape(N * 8))
    (a0, d4, gidx, sidx, wA1, wB1, wA2, wB2, wA3, wB3) = _tc0(
        src3, dst3, zsrc, pd_f.reshape(E, 4), edge_attr,
        w1r2, w1e2, b12, eye2, w1Lr, w1Le, radL_b1, radL_W2, radL_b2)
    gidx_f = gidx.reshape(E * 4)
    sidx_f = sidx.reshape(E * 4)
    d4_f = d4.reshape(E * 4)
    part0 = _sc_scatter0(sidx_f, a0.reshape(E * 4), zeros_pad)
    ones16 = jnp.ones((1, 16), _f32)
    t1 = _fin(part0, prev0, ones16)
    part1 = _sc_mid(gidx_f, sidx_f, d4_f, wA1.reshape(E * 4),
                    wB1.reshape(E * 4), t1.reshape(N * 4), zeros_pad)
    t2 = _fin(part1, t1, sivec(0))
    part2 = _sc_mid(gidx_f, sidx_f, d4_f, wA2.reshape(E * 4),
                    wB2.reshape(E * 4), t2.reshape(N * 4), zeros_pad)
    t3 = _fin(part2, t2, sivec(1))
    part3 = _sc_mid(gidx_f, sidx_f, d4_f, wA3.reshape(E * 4),
                    wB3.reshape(E * 4), t3.reshape(N * 4), zeros_pad)
    out = _fin(part3, t3, sivec(2))
    return out.reshape(N, 4)


# speedup vs baseline: 3.8881x; 1.0488x over previous
import jax
import jax.numpy as jnp
from jax import lax
from jax.experimental import pallas as pl
from jax.experimental.pallas import tpu as pltpu
from jax.experimental.pallas import tpu_sc as plsc

N = 10000
E = 320000
D = 128
EDIM = 16
HID = 32

NC = 2
NS = 16
NW = NC * NS
EPW = E // NW
CH = 400
NCH = EPW // CH
PAD = 40960

_f32 = jnp.float32
_i32 = jnp.int32

_MESH = dict(core_axis_name="c", subcore_axis_name="s")


def _prep_body(x_ref, pos_ref, wz_ref, ws_ref, ztab_ref, ptab_ref, prev0_ref):
    xb = x_ref[...]
    nb = xb.shape[0]
    z = jnp.dot(xb, wz_ref[...], preferred_element_type=_f32)
    y2 = jnp.dot(xb, ws_ref[...], preferred_element_type=_f32)
    posb = pos_ref[...]
    ztab_ref[...] = jnp.concatenate(
        [z, posb, y2[:, :2], jnp.zeros((nb, 59), _f32)], axis=1)
    ptab_ref[...] = jnp.concatenate([posb, y2[:, :5]], axis=1)
    prev0_ref[...] = jnp.concatenate(
        [y2[:, 2:3], jnp.zeros((nb, 3), _f32)], axis=1)


def _tc_prep(x, pos, wz, ws):
    blk = 2000
    grid = (N // blk,)
    return pl.pallas_call(
        _prep_body,
        grid=grid,
        in_specs=[
            pl.BlockSpec((blk, D), lambda i: (i, 0)),
            pl.BlockSpec((blk, 3), lambda i: (i, 0)),
            pl.BlockSpec((D, 64), lambda i: (0, 0)),
            pl.BlockSpec((D, 8), lambda i: (0, 0)),
        ],
        out_specs=[
            pl.BlockSpec((blk, 128), lambda i: (i, 0)),
            pl.BlockSpec((blk, 8), lambda i: (i, 0)),
            pl.BlockSpec((blk, 4), lambda i: (i, 0)),
        ],
        out_shape=[
            jax.ShapeDtypeStruct((N, 128), _f32),
            jax.ShapeDtypeStruct((N, 8), _f32),
            jax.ShapeDtypeStruct((N, 4), _f32),
        ],
    )(x, pos, wz, ws)


def _sc_gather0_body(src_hbm, pdidx_hbm, ztab_hbm, ptabf_hbm,
                     zsrc_out, pd_out,
                     src_v, pdidx_v, zbuf, pdbuf, sem1, sem2, sem3, sem4):
    c = lax.axis_index("c")
    s = lax.axis_index("s")
    base = (s * NC + c) * EPW

    def chunk(j, carry):
        off = pl.multiple_of(base + j * CH, 16)
        off4 = pl.multiple_of(off * 4, 64)
        ld1 = pltpu.async_copy(src_hbm.at[pl.ds(off, CH)], src_v, sem3)
        ld2 = pltpu.async_copy(pdidx_hbm.at[pl.ds(off4, CH * 4)], pdidx_v,
                               sem4)
        ld1.wait()
        cp1 = pltpu.async_copy(ztab_hbm.at[src_v], zbuf, sem1)
        ld2.wait()
        cp2 = pltpu.async_copy(ptabf_hbm.at[pdidx_v], pdbuf, sem2)
        cp1.wait()
        st1 = pltpu.async_copy(zbuf, zsrc_out.at[pl.ds(off, CH), :], sem3)
        cp2.wait()
        st2 = pltpu.async_copy(pdbuf, pd_out.at[pl.ds(off4, CH * 4)], sem4)
        st1.wait()
        st2.wait()
        return carry

    lax.fori_loop(0, NCH, chunk, 0)


def _sc_gather0(src, pdidx_f, ztab, ptab_f):
    return pl.kernel(
        _sc_gather0_body,
        out_type=(
            jax.ShapeDtypeStruct((E, 128), _f32),
            jax.ShapeDtypeStruct((E * 4,), _f32),
        ),
        mesh=plsc.VectorSubcoreMesh(**_MESH),
        scratch_types=[
            pltpu.VMEM((CH,), _i32),
            pltpu.VMEM((CH * 4,), _i32),
            pltpu.VMEM((CH, 128), _f32),
            pltpu.VMEM((CH * 4,), _f32),
            pltpu.SemaphoreType.DMA,
            pltpu.SemaphoreType.DMA,
            pltpu.SemaphoreType.DMA,
            pltpu.SemaphoreType.DMA,
        ],
    )(src, pdidx_f, ztab, ptab_f)


def _tc0_body(src_ref, dst_ref, zs_ref, pd_ref, ea_ref,
              w1r2_ref, w1e2_ref, b12_ref, msel_ref,
              w1Lr_ref, w1Le_ref, b1L_ref, w2L_ref, b2L_ref,
              a0_ref, d4_ref, gidx_ref, sidx_ref,
              wA1_ref, wB1_ref, wA2_ref, wB2_ref, wA3_ref, wB3_ref):
    zs = zs_ref[...]
    pd = pd_ref[...]
    rel = pd[:, 0:3] - zs[:, 64:67]
    c0s = zs[:, 67:68]
    c1s = zs[:, 68:69]
    ea = ea_ref[...]
    r2 = jnp.sum(rel * rel, axis=-1, keepdims=True)
    r = jnp.sqrt(r2)
    dirn = rel / (r + 1e-8)
    nb = rel.shape[0]

    hid2 = jnp.maximum(
        r * w1r2_ref[...] + jnp.dot(ea, w1e2_ref[...],
                                    preferred_element_type=_f32)
        + b12_ref[...], 0.0)
    prod = hid2 * zs[:, 0:64]
    ms = jnp.dot(prod, msel_ref[...], preferred_element_type=_f32)
    m0 = ms[:, 0:1] + c0s
    s01 = ms[:, 1:2] + c1s
    a0_ref[...] = jnp.concatenate([m0, s01 * dirn], axis=1)

    d4_ref[...] = jnp.concatenate([jnp.zeros((nb, 1), _f32), dirn], axis=1)
    lane = lax.broadcasted_iota(_i32, (nb, 4), 1)
    gidx_ref[...] = src_ref[0, 0, :].reshape(nb, 1) * 4 + lane
    sidx_ref[...] = dst_ref[0, 0, :].reshape(nb, 1) * 4 + lane

    for i, (wA_ref, wB_ref) in enumerate(
            ((wA1_ref, wB1_ref), (wA2_ref, wB2_ref), (wA3_ref, wB3_ref))):
        hidL = jnp.maximum(
            r * w1Lr_ref[i][None, :]
            + jnp.dot(ea, w1Le_ref[i], preferred_element_type=_f32)
            + b1L_ref[i][None, :], 0.0)
        feat = (jnp.dot(hidL, w2L_ref[i], preferred_element_type=_f32)
                + b2L_ref[i][None, :])
        f00 = feat[:, 0:1]
        f01 = feat[:, 1:2]
        f10 = feat[:, 2:3]
        f11 = feat[:, 3:4]
        wA_ref[...] = jnp.concatenate([f00, f11, f11, f11], axis=1)
        wB_ref[...] = jnp.concatenate([f10, f01, f01, f01], axis=1)


def _tc0(src3, dst3, zsrc, pd4, ea,
         w1r2, w1e2, b12, msel, w1Lr, w1Le, b1L, w2L, b2L):
    blk = 2560
    grid = (E // blk,)
    full = lambda *dims: pl.BlockSpec(dims, lambda i: tuple(0 for _ in dims))
    eblk4 = pl.BlockSpec((blk, 4), lambda i: (i, 0))
    out4 = jax.ShapeDtypeStruct((E, 4), _f32)
    out4i = jax.ShapeDtypeStruct((E, 4), _i32)
    return pl.pallas_call(
        _tc0_body,
        grid=grid,
        in_specs=[
            pl.BlockSpec((1, 1, blk), lambda i: (i, 0, 0)),
            pl.BlockSpec((1, 1, blk), lambda i: (i, 0, 0)),
            pl.BlockSpec((blk, 128), lambda i: (i, 0)),
            pl.BlockSpec((blk, 4), lambda i: (i, 0)),
            pl.BlockSpec((blk, EDIM), lambda i: (i, 0)),
            full(1, 64), full(EDIM, 64), full(1, 64),
            full(64, 8),
            full(3, HID), full(3, EDIM, HID), full(3, HID),
            full(3, HID, 4), full(3, 4),
        ],
        out_specs=[eblk4] * 10,
        out_shape=[out4, out4, out4i, out4i] + [out4] * 6,
    )(src3, dst3, zsrc, pd4, ea,
      w1r2, w1e2, b12, msel, w1Lr, w1Le, b1L, w2L, b2L)


def _sc_scatter0_body(sidx_hbm, a0_hbm, zeros_hbm, part_out,
                      dst_v, msg_v, acc):
    c = lax.axis_index("c")
    s = lax.axis_index("s")
    base4 = (s * NC + c) * (EPW * 4)
    soff = pl.multiple_of(s * PAD, 128)

    pltpu.sync_copy(zeros_hbm, acc.at[pl.ds(soff, PAD)])

    def chunk(j, carry):
        off4 = pl.multiple_of(base4 + j * (CH * 4), 64)
        pltpu.sync_copy(sidx_hbm.at[pl.ds(off4, CH * 4)], dst_v)
        pltpu.sync_copy(a0_hbm.at[pl.ds(off4, CH * 4)], msg_v)
        for t in range(CH * 4 // 16):
            o = t * 16
            dst_v[pl.ds(o, 16)] = dst_v[pl.ds(o, 16)] + soff
        pltpu.sync_copy(msg_v, acc.at[dst_v], add=True)
        return carry

    lax.fori_loop(0, NCH, chunk, 0)
    pltpu.sync_copy(acc.at[pl.ds(soff, PAD)],
                    part_out.at[c, pl.ds(soff, PAD)])


def _sc_scatter0(sidx_f, a0_f, zeros_pad):
    return pl.kernel(
        _sc_scatter0_body,
        out_type=jax.ShapeDtypeStruct((NC, NS * PAD), _f32),
        mesh=plsc.VectorSubcoreMesh(**_MESH),
        scratch_types=[
            pltpu.VMEM((CH * 4,), _i32),
            pltpu.VMEM((CH * 4,), _f32),
            pltpu.VMEM_SHARED((NS * PAD,), _f32),
        ],
    )(sidx_f, a0_f, zeros_pad)


def _sc_mid_body(gidx_hbm, sidx_hbm, d4_hbm, wA_hbm, wB_hbm, t4_hbm,
                 zeros_hbm, part_out,
                 gidx_v, sidx_v, d4_v, wA_v, wB_v, hs_v, msg_v, acc, sem,
                 sem1, sem2, sem3, sem4, sem5):
    c = lax.axis_index("c")
    s = lax.axis_index("s")
    base4 = (s * NC + c) * (EPW * 4)
    soff = pl.multiple_of(s * PAD, 128)

    pltpu.sync_copy(zeros_hbm, acc.at[pl.ds(soff, PAD)])

    lanes = lax.iota(_i32, 16)
    e0 = (lanes & 3) == 0
    permb = lanes ^ 1
    permq = lanes ^ 2
    permh = (lanes >> 2) * 4

    def chunk(j, carry):
        off4 = pl.multiple_of(base4 + j * (CH * 4), 64)
        ld1 = pltpu.async_copy(gidx_hbm.at[pl.ds(off4, CH * 4)], gidx_v,
                               sem1)
        ld2 = pltpu.async_copy(sidx_hbm.at[pl.ds(off4, CH * 4)], sidx_v,
                               sem2)
        ld3 = pltpu.async_copy(d4_hbm.at[pl.ds(off4, CH * 4)], d4_v, sem3)
        ld4 = pltpu.async_copy(wA_hbm.at[pl.ds(off4, CH * 4)], wA_v, sem4)
        ld5 = pltpu.async_copy(wB_hbm.at[pl.ds(off4, CH * 4)], wB_v, sem5)
        ld1.wait()
        cpg = pltpu.async_copy(t4_hbm.at[gidx_v], hs_v, sem)
        ld2.wait()
        ld3.wait()
        ld4.wait()
        ld5.wait()
        cpg.wait()
        for t in range(CH * 4 // 16):
            o = t * 16
            hs = hs_v[pl.ds(o, 16)]
            d4 = d4_v[pl.ds(o, 16)]
            wa = wA_v[pl.ds(o, 16)]
            wb = wB_v[pl.ds(o, 16)]
            p = d4 * hs
            q = p + p[permb]
            dotv = q + q[permq]
            h0b = hs[permh]
            m4 = jnp.where(e0, dotv, h0b * d4)
            msg_v[pl.ds(o, 16)] = wa * hs + wb * m4
            sidx_v[pl.ds(o, 16)] = sidx_v[pl.ds(o, 16)] + soff
        pltpu.sync_copy(msg_v, acc.at[sidx_v], add=True)
        return carry

    lax.fori_loop(0, NCH, chunk, 0)
    pltpu.sync_copy(acc.at[pl.ds(soff, PAD)],
                    part_out.at[c, pl.ds(soff, PAD)])


def _sc_mid(gidx, sidx, d4, wA, wB, t4, zeros_pad):
    return pl.kernel(
        _sc_mid_body,
        out_type=jax.ShapeDtypeStruct((NC, NS * PAD), _f32),
        mesh=plsc.VectorSubcoreMesh(**_MESH),
        scratch_types=[
            pltpu.VMEM((CH * 4,), _i32),
            pltpu.VMEM((CH * 4,), _i32),
            pltpu.VMEM((CH * 4,), _f32),
            pltpu.VMEM((CH * 4,), _f32),
            pltpu.VMEM((CH * 4,), _f32),
            pltpu.VMEM((CH * 4,), _f32),
            pltpu.VMEM((CH * 4,), _f32),
            pltpu.VMEM_SHARED((NS * PAD,), _f32),
            pltpu.SemaphoreType.DMA,
            pltpu.SemaphoreType.DMA,
            pltpu.SemaphoreType.DMA,
            pltpu.SemaphoreType.DMA,
            pltpu.SemaphoreType.DMA,
            pltpu.SemaphoreType.DMA,
        ],
    )(gidx, sidx, d4, wA, wB, t4, zeros_pad)


def _fin_body(p_ref, prev_ref, si_ref, out_ref):
    out_ref[...] = (jnp.sum(p_ref[...], axis=0)
                    + prev_ref[...] * si_ref[...])


def _fin(part, prev, si16):
    rows = N // 4
    p32 = part.reshape(NW, PAD)[:, :N * 4].reshape(NW, rows, 16)
    return pl.pallas_call(
        _fin_body,
        out_shape=jax.ShapeDtypeStruct((rows, 16), _f32),
    )(p32, prev.reshape(rows, 16), si16)


def _jnp_gather0(src, pdidx_f, ztab, ptab_f):
    return ztab[src], ptab_f[pdidx_f]


def _jnp_scatter0(dst, a0, zeros_nt):
    from jax.ops import segment_sum
    seg = segment_sum(a0, dst, num_segments=N)
    return jnp.stack([seg, jnp.zeros_like(seg)])


def _jnp_mid(gidx, sidx, d4, wA, wB, t4, zeros_flat):
    hs = t4[gidx].reshape(E, 4)
    d = d4.reshape(E, 4)
    wa = wA.reshape(E, 4)
    wb = wB.reshape(E, 4)
    dot = (d * hs).sum(-1, keepdims=True)
    m4 = jnp.concatenate([dot, hs[:, 0:1] * d[:, 1:]], axis=1)
    msg = (wa * hs + wb * m4).reshape(-1)
    accf = jnp.zeros((N * 4,), _f32).at[sidx].add(msg)
    return jnp.stack([accf, jnp.zeros_like(accf)])


def kernel(x, pos, edge_index, edge_attr, rad0_W1, rad0_b1, rad0_W2,
           rad0_b2, si0_0, radL_W1, radL_b1, radL_W2, radL_b2, siL_0, siL_1):
    src = edge_index[0]
    dst = edge_index[1]
    blk = 2560
    src3 = src.reshape(E // blk, 1, blk)
    dst3 = dst.reshape(E // blk, 1, blk)

    wz = jnp.concatenate([rad0_W2[:, :D].T, rad0_W2[:, D:].T], axis=1)
    ws = jnp.stack(
        [rad0_b2[:D], rad0_b2[D:], si0_0[0]]
        + [jnp.zeros((D,), _f32)] * 5, axis=1)
    w1r2 = jnp.concatenate([rad0_W1[0:1], rad0_W1[0:1]], axis=1)
    w1e2 = jnp.concatenate([rad0_W1[1:], rad0_W1[1:]], axis=1)
    b12 = jnp.concatenate([rad0_b1, rad0_b1]).reshape(1, 64)
    eye2 = jnp.zeros((64, 8), _f32)
    eye2 = eye2.at[:HID, 0].set(1.0).at[HID:, 1].set(1.0)
    w1Lr = radL_W1[:, 0, :]
    w1Le = radL_W1[:, 1:, :]
    zeros_pad = jnp.zeros((PAD,), _f32)

    def sivec(i):
        v4 = jnp.concatenate([siL_0[i, 0], siL_1[i, 0], siL_1[i, 0],
                              siL_1[i, 0]])
        return jnp.tile(v4, 4).reshape(1, 16)

    ztab, ptab, prev0 = _tc_prep(x, pos, wz, ws)
    pdidx_f = (dst[:, None] * 8 + jnp.arange(4, dtype=_i32)).reshape(E * 4)
    zsrc, pd_f = _sc_gather0(src, pdidx_f, ztab, ptab.reshape(N * 8))
    (a0, d4, gidx, sidx, wA1, wB1, wA2, wB2, wA3, wB3) = _tc0(
        src3, dst3, zsrc, pd_f.reshape(E, 4), edge_attr,
        w1r2, w1e2, b12, eye2, w1Lr, w1Le, radL_b1, radL_W2, radL_b2)
    gidx_f = gidx.reshape(E * 4)
    sidx_f = sidx.reshape(E * 4)
    d4_f = d4.reshape(E * 4)
    part0 = _sc_scatter0(sidx_f, a0.reshape(E * 4), zeros_pad)
    ones16 = jnp.ones((1, 16), _f32)
    t1 = _fin(part0, prev0, ones16)
    part1 = _sc_mid(gidx_f, sidx_f, d4_f, wA1.reshape(E * 4),
                    wB1.reshape(E * 4), t1.reshape(N * 4), zeros_pad)
    t2 = _fin(part1, t1, sivec(0))
    part2 = _sc_mid(gidx_f, sidx_f, d4_f, wA2.reshape(E * 4),
                    wB2.reshape(E * 4), t2.reshape(N * 4), zeros_pad)
    t3 = _fin(part2, t2, sivec(1))
    part3 = _sc_mid(gidx_f, sidx_f, d4_f, wA3.reshape(E * 4),
                    wB3.reshape(E * 4), t3.reshape(N * 4), zeros_pad)
    out = _fin(part3, t3, sivec(2))
    return out.reshape(N, 4)


# speedup vs baseline: 3.9427x; 1.0140x over previous
import jax
import jax.numpy as jnp
from jax import lax
from jax.experimental import pallas as pl
from jax.experimental.pallas import tpu as pltpu
from jax.experimental.pallas import tpu_sc as plsc

N = 10000
E = 320000
D = 128
EDIM = 16
HID = 32

NC = 2
NS = 16
NW = NC * NS
EPW = E // NW
CH = 400
NCH = EPW // CH
CHM = 1000
NCHM = EPW // CHM
PAD = 40960

_f32 = jnp.float32
_i32 = jnp.int32

_MESH = dict(core_axis_name="c", subcore_axis_name="s")


def _prep_body(x_ref, pos_ref, wz_ref, ws_ref, ztab_ref, ptab_ref, prev0_ref):
    xb = x_ref[...]
    nb = xb.shape[0]
    z = jnp.dot(xb, wz_ref[...], preferred_element_type=_f32)
    y2 = jnp.dot(xb, ws_ref[...], preferred_element_type=_f32)
    posb = pos_ref[...]
    ztab_ref[...] = jnp.concatenate(
        [z, posb, y2[:, :2], jnp.zeros((nb, 59), _f32)], axis=1)
    ptab_ref[...] = jnp.concatenate([posb, y2[:, :5]], axis=1)
    prev0_ref[...] = jnp.concatenate(
        [y2[:, 2:3], jnp.zeros((nb, 3), _f32)], axis=1)


def _tc_prep(x, pos, wz, ws):
    blk = 2000
    grid = (N // blk,)
    return pl.pallas_call(
        _prep_body,
        grid=grid,
        in_specs=[
            pl.BlockSpec((blk, D), lambda i: (i, 0)),
            pl.BlockSpec((blk, 3), lambda i: (i, 0)),
            pl.BlockSpec((D, 64), lambda i: (0, 0)),
            pl.BlockSpec((D, 8), lambda i: (0, 0)),
        ],
        out_specs=[
            pl.BlockSpec((blk, 128), lambda i: (i, 0)),
            pl.BlockSpec((blk, 8), lambda i: (i, 0)),
            pl.BlockSpec((blk, 4), lambda i: (i, 0)),
        ],
        out_shape=[
            jax.ShapeDtypeStruct((N, 128), _f32),
            jax.ShapeDtypeStruct((N, 8), _f32),
            jax.ShapeDtypeStruct((N, 4), _f32),
        ],
    )(x, pos, wz, ws)


def _sc_gather0_body(src_hbm, pdidx_hbm, ztab_hbm, ptabf_hbm,
                     zsrc_out, pd_out,
                     src_v, pdidx_v, zbuf, pdbuf, sem1, sem2, sem3, sem4):
    c = lax.axis_index("c")
    s = lax.axis_index("s")
    base = (s * NC + c) * EPW

    def chunk(j, carry):
        off = pl.multiple_of(base + j * CH, 16)
        off4 = pl.multiple_of(off * 4, 64)
        ld1 = pltpu.async_copy(src_hbm.at[pl.ds(off, CH)], src_v, sem3)
        ld2 = pltpu.async_copy(pdidx_hbm.at[pl.ds(off4, CH * 4)], pdidx_v,
                               sem4)
        ld1.wait()
        cp1 = pltpu.async_copy(ztab_hbm.at[src_v], zbuf, sem1)
        ld2.wait()
        cp2 = pltpu.async_copy(ptabf_hbm.at[pdidx_v], pdbuf, sem2)
        cp1.wait()
        st1 = pltpu.async_copy(zbuf, zsrc_out.at[pl.ds(off, CH), :], sem3)
        cp2.wait()
        st2 = pltpu.async_copy(pdbuf, pd_out.at[pl.ds(off4, CH * 4)], sem4)
        st1.wait()
        st2.wait()
        return carry

    lax.fori_loop(0, NCH, chunk, 0)


def _sc_gather0(src, pdidx_f, ztab, ptab_f):
    return pl.kernel(
        _sc_gather0_body,
        out_type=(
            jax.ShapeDtypeStruct((E, 128), _f32),
            jax.ShapeDtypeStruct((E * 4,), _f32),
        ),
        mesh=plsc.VectorSubcoreMesh(**_MESH),
        scratch_types=[
            pltpu.VMEM((CH,), _i32),
            pltpu.VMEM((CH * 4,), _i32),
            pltpu.VMEM((CH, 128), _f32),
            pltpu.VMEM((CH * 4,), _f32),
            pltpu.SemaphoreType.DMA,
            pltpu.SemaphoreType.DMA,
            pltpu.SemaphoreType.DMA,
            pltpu.SemaphoreType.DMA,
        ],
    )(src, pdidx_f, ztab, ptab_f)


def _tc0_body(src_ref, dst_ref, zs_ref, pd_ref, ea_ref,
              w1r2_ref, w1e2_ref, b12_ref, msel_ref,
              w1Lr_ref, w1Le_ref, b1L_ref, w2L_ref, b2L_ref,
              a0_ref, d4_ref, gidx_ref, sidx_ref,
              wA1_ref, wB1_ref, wA2_ref, wB2_ref, wA3_ref, wB3_ref):
    zs = zs_ref[...]
    pd = pd_ref[...]
    rel = pd[:, 0:3] - zs[:, 64:67]
    c0s = zs[:, 67:68]
    c1s = zs[:, 68:69]
    ea = ea_ref[...]
    r2 = jnp.sum(rel * rel, axis=-1, keepdims=True)
    r = jnp.sqrt(r2)
    dirn = rel / (r + 1e-8)
    nb = rel.shape[0]

    hid2 = jnp.maximum(
        r * w1r2_ref[...] + jnp.dot(ea, w1e2_ref[...],
                                    preferred_element_type=_f32)
        + b12_ref[...], 0.0)
    prod = hid2 * zs[:, 0:64]
    ms = jnp.dot(prod, msel_ref[...], preferred_element_type=_f32)
    m0 = ms[:, 0:1] + c0s
    s01 = ms[:, 1:2] + c1s
    a0_ref[...] = jnp.concatenate([m0, s01 * dirn], axis=1)

    d4_ref[...] = jnp.concatenate([jnp.zeros((nb, 1), _f32), dirn], axis=1)
    lane = lax.broadcasted_iota(_i32, (nb, 4), 1)
    gidx_ref[...] = src_ref[0, 0, :].reshape(nb, 1) * 4 + lane
    sidx_ref[...] = dst_ref[0, 0, :].reshape(nb, 1) * 4 + lane

    for i, (wA_ref, wB_ref) in enumerate(
            ((wA1_ref, wB1_ref), (wA2_ref, wB2_ref), (wA3_ref, wB3_ref))):
        hidL = jnp.maximum(
            r * w1Lr_ref[i][None, :]
            + jnp.dot(ea, w1Le_ref[i], preferred_element_type=_f32)
            + b1L_ref[i][None, :], 0.0)
        feat = (jnp.dot(hidL, w2L_ref[i], preferred_element_type=_f32)
                + b2L_ref[i][None, :])
        f00 = feat[:, 0:1]
        f01 = feat[:, 1:2]
        f10 = feat[:, 2:3]
        f11 = feat[:, 3:4]
        wA_ref[...] = jnp.concatenate([f00, f11, f11, f11], axis=1)
        wB_ref[...] = jnp.concatenate([f10, f01, f01, f01], axis=1)


def _tc0(src3, dst3, zsrc, pd4, ea,
         w1r2, w1e2, b12, msel, w1Lr, w1Le, b1L, w2L, b2L):
    blk = 2560
    grid = (E // blk,)
    full = lambda *dims: pl.BlockSpec(dims, lambda i: tuple(0 for _ in dims))
    eblk4 = pl.BlockSpec((blk, 4), lambda i: (i, 0))
    out4 = jax.ShapeDtypeStruct((E, 4), _f32)
    out4i = jax.ShapeDtypeStruct((E, 4), _i32)
    return pl.pallas_call(
        _tc0_body,
        grid=grid,
        in_specs=[
            pl.BlockSpec((1, 1, blk), lambda i: (i, 0, 0)),
            pl.BlockSpec((1, 1, blk), lambda i: (i, 0, 0)),
            pl.BlockSpec((blk, 128), lambda i: (i, 0)),
            pl.BlockSpec((blk, 4), lambda i: (i, 0)),
            pl.BlockSpec((blk, EDIM), lambda i: (i, 0)),
            full(1, 64), full(EDIM, 64), full(1, 64),
            full(64, 8),
            full(3, HID), full(3, EDIM, HID), full(3, HID),
            full(3, HID, 4), full(3, 4),
        ],
        out_specs=[eblk4] * 10,
        out_shape=[out4, out4, out4i, out4i] + [out4] * 6,
    )(src3, dst3, zsrc, pd4, ea,
      w1r2, w1e2, b12, msel, w1Lr, w1Le, b1L, w2L, b2L)


def _sc_scatter0_body(sidx_hbm, a0_hbm, zeros_hbm, part_out,
                      dst_v, msg_v, acc):
    c = lax.axis_index("c")
    s = lax.axis_index("s")
    base4 = (s * NC + c) * (EPW * 4)
    soff = pl.multiple_of(s * PAD, 128)

    pltpu.sync_copy(zeros_hbm, acc.at[pl.ds(soff, PAD)])

    def chunk(j, carry):
        off4 = pl.multiple_of(base4 + j * (CHM * 4), 64)
        pltpu.sync_copy(sidx_hbm.at[pl.ds(off4, CHM * 4)], dst_v)
        pltpu.sync_copy(a0_hbm.at[pl.ds(off4, CHM * 4)], msg_v)
        for t in range(CHM * 4 // 16):
            o = t * 16
            dst_v[pl.ds(o, 16)] = dst_v[pl.ds(o, 16)] + soff
        pltpu.sync_copy(msg_v, acc.at[dst_v], add=True)
        return carry

    lax.fori_loop(0, NCHM, chunk, 0)
    pltpu.sync_copy(acc.at[pl.ds(soff, PAD)],
                    part_out.at[c, pl.ds(soff, PAD)])


def _sc_scatter0(sidx_f, a0_f, zeros_pad):
    return pl.kernel(
        _sc_scatter0_body,
        out_type=jax.ShapeDtypeStruct((NC, NS * PAD), _f32),
        mesh=plsc.VectorSubcoreMesh(**_MESH),
        scratch_types=[
            pltpu.VMEM((CHM * 4,), _i32),
            pltpu.VMEM((CHM * 4,), _f32),
            pltpu.VMEM_SHARED((NS * PAD,), _f32),
        ],
    )(sidx_f, a0_f, zeros_pad)


def _sc_mid_body(gidx_hbm, sidx_hbm, d4_hbm, wA_hbm, wB_hbm, t4_hbm,
                 zeros_hbm, part_out,
                 gidx_v, sidx_v, d4_v, wA_v, wB_v, hs_v, msg_v, acc, stbl,
                 sem, sem1, sem2, sem3, sem4, sem5):
    c = lax.axis_index("c")
    s = lax.axis_index("s")
    base4 = (s * NC + c) * (EPW * 4)
    soff = pl.multiple_of(s * PAD, 128)

    pltpu.sync_copy(zeros_hbm, acc.at[pl.ds(soff, PAD)])

    @pl.when(s == 0)
    def _():
        pltpu.sync_copy(t4_hbm, stbl)

    plsc.subcore_barrier()

    lanes = lax.iota(_i32, 16)
    e0 = (lanes & 3) == 0
    permb = lanes ^ 1
    permq = lanes ^ 2
    permh = (lanes >> 2) * 4

    def chunk(j, carry):
        off4 = pl.multiple_of(base4 + j * (CHM * 4), 64)
        ld1 = pltpu.async_copy(gidx_hbm.at[pl.ds(off4, CHM * 4)], gidx_v,
                               sem1)
        ld2 = pltpu.async_copy(sidx_hbm.at[pl.ds(off4, CHM * 4)], sidx_v,
                               sem2)
        ld3 = pltpu.async_copy(d4_hbm.at[pl.ds(off4, CHM * 4)], d4_v, sem3)
        ld4 = pltpu.async_copy(wA_hbm.at[pl.ds(off4, CHM * 4)], wA_v, sem4)
        ld5 = pltpu.async_copy(wB_hbm.at[pl.ds(off4, CHM * 4)], wB_v, sem5)
        ld1.wait()
        cpg = pltpu.async_copy(stbl.at[gidx_v], hs_v, sem)
        ld2.wait()
        ld3.wait()
        ld4.wait()
        ld5.wait()
        cpg.wait()
        for t in range(CHM * 4 // 16):
            o = t * 16
            hs = hs_v[pl.ds(o, 16)]
            d4 = d4_v[pl.ds(o, 16)]
            wa = wA_v[pl.ds(o, 16)]
            wb = wB_v[pl.ds(o, 16)]
            p = d4 * hs
            q = p + p[permb]
            dotv = q + q[permq]
            h0b = hs[permh]
            m4 = jnp.where(e0, dotv, h0b * d4)
            msg_v[pl.ds(o, 16)] = wa * hs + wb * m4
            sidx_v[pl.ds(o, 16)] = sidx_v[pl.ds(o, 16)] + soff
        pltpu.sync_copy(msg_v, acc.at[sidx_v], add=True)
        return carry

    lax.fori_loop(0, NCHM, chunk, 0)
    pltpu.sync_copy(acc.at[pl.ds(soff, PAD)],
                    part_out.at[c, pl.ds(soff, PAD)])


def _sc_mid(gidx, sidx, d4, wA, wB, t4, zeros_pad):
    return pl.kernel(
        _sc_mid_body,
        out_type=jax.ShapeDtypeStruct((NC, NS * PAD), _f32),
        mesh=plsc.VectorSubcoreMesh(**_MESH),
        scratch_types=[
            pltpu.VMEM((CHM * 4,), _i32),
            pltpu.VMEM((CHM * 4,), _i32),
            pltpu.VMEM((CHM * 4,), _f32),
            pltpu.VMEM((CHM * 4,), _f32),
            pltpu.VMEM((CHM * 4,), _f32),
            pltpu.VMEM((CHM * 4,), _f32),
            pltpu.VMEM((CHM * 4,), _f32),
            pltpu.VMEM_SHARED((NS * PAD,), _f32),
            pltpu.VMEM_SHARED((N * 4,), _f32),
            pltpu.SemaphoreType.DMA,
            pltpu.SemaphoreType.DMA,
            pltpu.SemaphoreType.DMA,
            pltpu.SemaphoreType.DMA,
            pltpu.SemaphoreType.DMA,
            pltpu.SemaphoreType.DMA,
        ],
    )(gidx, sidx, d4, wA, wB, t4, zeros_pad)


def _fin_body(p_ref, prev_ref, si_ref, out_ref):
    out_ref[...] = (jnp.sum(p_ref[...], axis=0)
                    + prev_ref[...] * si_ref[...])


def _fin(part, prev, si16):
    rows = N // 4
    p32 = part.reshape(NW, PAD)[:, :N * 4].reshape(NW, rows, 16)
    return pl.pallas_call(
        _fin_body,
        out_shape=jax.ShapeDtypeStruct((rows, 16), _f32),
    )(p32, prev.reshape(rows, 16), si16)


def _jnp_gather0(src, pdidx_f, ztab, ptab_f):
    return ztab[src], ptab_f[pdidx_f]


def _jnp_scatter0(dst, a0, zeros_nt):
    from jax.ops import segment_sum
    seg = segment_sum(a0, dst, num_segments=N)
    return jnp.stack([seg, jnp.zeros_like(seg)])


def _jnp_mid(gidx, sidx, d4, wA, wB, t4, zeros_flat):
    hs = t4[gidx].reshape(E, 4)
    d = d4.reshape(E, 4)
    wa = wA.reshape(E, 4)
    wb = wB.reshape(E, 4)
    dot = (d * hs).sum(-1, keepdims=True)
    m4 = jnp.concatenate([dot, hs[:, 0:1] * d[:, 1:]], axis=1)
    msg = (wa * hs + wb * m4).reshape(-1)
    accf = jnp.zeros((N * 4,), _f32).at[sidx].add(msg)
    return jnp.stack([accf, jnp.zeros_like(accf)])


def kernel(x, pos, edge_index, edge_attr, rad0_W1, rad0_b1, rad0_W2,
           rad0_b2, si0_0, radL_W1, radL_b1, radL_W2, radL_b2, siL_0, siL_1):
    src = edge_index[0]
    dst = edge_index[1]
    blk = 2560
    src3 = src.reshape(E // blk, 1, blk)
    dst3 = dst.reshape(E // blk, 1, blk)

    wz = jnp.concatenate([rad0_W2[:, :D].T, rad0_W2[:, D:].T], axis=1)
    ws = jnp.stack(
        [rad0_b2[:D], rad0_b2[D:], si0_0[0]]
        + [jnp.zeros((D,), _f32)] * 5, axis=1)
    w1r2 = jnp.concatenate([rad0_W1[0:1], rad0_W1[0:1]], axis=1)
    w1e2 = jnp.concatenate([rad0_W1[1:], rad0_W1[1:]], axis=1)
    b12 = jnp.concatenate([rad0_b1, rad0_b1]).reshape(1, 64)
    eye2 = jnp.zeros((64, 8), _f32)
    eye2 = eye2.at[:HID, 0].set(1.0).at[HID:, 1].set(1.0)
    w1Lr = radL_W1[:, 0, :]
    w1Le = radL_W1[:, 1:, :]
    zeros_pad = jnp.zeros((PAD,), _f32)

    def sivec(i):
        v4 = jnp.concatenate([siL_0[i, 0], siL_1[i, 0], siL_1[i, 0],
                              siL_1[i, 0]])
        return jnp.tile(v4, 4).reshape(1, 16)

    ztab, ptab, prev0 = _tc_prep(x, pos, wz, ws)
    pdidx_f = (dst[:, None] * 8 + jnp.arange(4, dtype=_i32)).reshape(E * 4)
    zsrc, pd_f = _sc_gather0(src, pdidx_f, ztab, ptab.reshape(N * 8))
    (a0, d4, gidx, sidx, wA1, wB1, wA2, wB2, wA3, wB3) = _tc0(
        src3, dst3, zsrc, pd_f.reshape(E, 4), edge_attr,
        w1r2, w1e2, b12, eye2, w1Lr, w1Le, radL_b1, radL_W2, radL_b2)
    gidx_f = gidx.reshape(E * 4)
    sidx_f = sidx.reshape(E * 4)
    d4_f = d4.reshape(E * 4)
    part0 = _sc_scatter0(sidx_f, a0.reshape(E * 4), zeros_pad)
    ones16 = jnp.ones((1, 16), _f32)
    t1 = _fin(part0, prev0, ones16)
    part1 = _sc_mid(gidx_f, sidx_f, d4_f, wA1.reshape(E * 4),
                    wB1.reshape(E * 4), t1.reshape(N * 4), zeros_pad)
    t2 = _fin(part1, t1, sivec(0))
    part2 = _sc_mid(gidx_f, sidx_f, d4_f, wA2.reshape(E * 4),
                    wB2.reshape(E * 4), t2.reshape(N * 4), zeros_pad)
    t3 = _fin(part2, t2, sivec(1))
    part3 = _sc_mid(gidx_f, sidx_f, d4_f, wA3.reshape(E * 4),
                    wB3.reshape(E * 4), t3.reshape(N * 4), zeros_pad)
    out = _fin(part3, t3, sivec(2))
    return out.reshape(N, 4)
